# baseline (speedup 1.0000x reference)
import functools

import jax
import jax.numpy as jnp
from jax import lax
from jax.experimental import pallas as pl
from jax.experimental.pallas import tpu as pltpu
from jax.experimental.pallas import tpu_sc as plsc

_EPS = 1e-5
_K = 20
_KPAD = 32
_NEG = -3.0e38
_NC = 2
_NS = 16
_NW = _NC * _NS
_GRP = 4


def _topk_kernel(P, K, ptsR_ref, ptsT_ref, wdT_ref, sgn_ref,
                 idx_ref, s_ref):
    b = pl.program_id(0)
    Xr = ptsR_ref[0]
    XT = ptsT_ref[0]
    R = Xr.shape[0]

    S = (jnp.dot(Xr, wdT_ref[...], preferred_element_type=jnp.float32)
         * sgn_ref[...])
    s_ref[0] = jnp.concatenate([S, jnp.zeros_like(S)], axis=1)

    rn = jnp.sum(Xr * Xr, axis=1, keepdims=True)
    cn = jnp.sum(XT * XT, axis=0, keepdims=True)
    D = 2.0 * jnp.dot(Xr, XT, preferred_element_type=jnp.float32) - rn - cn
    iota = lax.broadcasted_iota(jnp.int32, (R, P), 1)
    lane32 = lax.broadcasted_iota(jnp.int32, (R, _KPAD), 1)

    v0 = jnp.max(D, axis=1, keepdims=True)

    def body(i, carry):
        v_prev, idxs = carry
        cand = jnp.where(D == v_prev, iota, P)
        idx = jnp.min(cand, axis=1, keepdims=True)
        m = jnp.where(D >= v_prev, _NEG, D)
        v = jnp.max(m, axis=1, keepdims=True)
        idxs = jnp.where(lane32 == i - 1, idx, idxs)
        return v, idxs

    _, idxs = lax.fori_loop(1, K + 1,
                            body, (v0, jnp.zeros((R, _KPAD), jnp.int32)))
    idxs = jnp.where(lane32 < K, idxs, idxs[:, 0:1]) + b * P
    idx_ref[0] = idxs


def _run_topk(pts, wdT, sgn, row_tile):
    B, P, _ = pts.shape
    nT = P // row_tile
    ptsT = jnp.swapaxes(pts, 1, 2)
    kern = functools.partial(_topk_kernel, P, _K)
    return pl.pallas_call(
        kern,
        grid=(B, nT),
        in_specs=[
            pl.BlockSpec((1, row_tile, 4), lambda b, t: (b, t, 0)),
            pl.BlockSpec((1, 4, P), lambda b, t: (b, 0, 0)),
            pl.BlockSpec((4, 64), lambda b, t: (0, 0)),
            pl.BlockSpec((1, 64), lambda b, t: (0, 0)),
        ],
        out_specs=[
            pl.BlockSpec((1, row_tile, _KPAD), lambda b, t: (b, t, 0)),
            pl.BlockSpec((1, row_tile, 128), lambda b, t: (b, t, 0)),
        ],
        out_shape=[
            jax.ShapeDtypeStruct((B, P, _KPAD), jnp.int32),
            jax.ShapeDtypeStruct((B, P, 128), jnp.float32),
        ],
    )(pts, ptsT, wdT, sgn)


def _make_sc_gather_max(P):
    per_w = P // _NW
    ngrp = per_w // _GRP
    mesh = plsc.VectorSubcoreMesh(core_axis_name="c", subcore_axis_name="s")

    @functools.partial(
        pl.kernel, mesh=mesh,
        out_type=jax.ShapeDtypeStruct((P, 128), jnp.float32),
        scratch_types=[
            pltpu.VMEM((_GRP * _KPAD,), jnp.int32),
            pltpu.VMEM((_GRP * _KPAD,), jnp.int32),
            pltpu.VMEM((_GRP * _KPAD, 128), jnp.float32),
            pltpu.VMEM((_GRP * _KPAD, 128), jnp.float32),
            pltpu.VMEM((_GRP, 128), jnp.float32),
            pltpu.SemaphoreType.DMA,
            pltpu.SemaphoreType.DMA,
        ],
    )
    def sc_kernel(s_hbm, idx_hbm, out_hbm, idx_v0, idx_v1,
                  rows_v0, rows_v1, out_v, sem0, sem1):
        wid = lax.axis_index("s") * _NC + lax.axis_index("c")
        base_pt = wid * per_w
        idx_vs = (idx_v0, idx_v1)
        rows_vs = (rows_v0, rows_v1)
        sems = (sem0, sem1)

        def stage(g, slot):
            gbase = base_pt + g * _GRP
            pltpu.sync_copy(idx_hbm.at[pl.ds(gbase * _KPAD, _GRP * _KPAD)],
                            idx_vs[slot])
            pltpu.async_copy(s_hbm.at[idx_vs[slot]], rows_vs[slot],
                             sems[slot])

        def compute(g, slot):
            gbase = base_pt + g * _GRP
            pltpu.make_async_copy(s_hbm.at[idx_vs[slot]], rows_vs[slot],
                                  sems[slot]).wait()
            rows_v = rows_vs[slot]
            zero = jnp.zeros((16,), jnp.float32)
            for q in range(_GRP):
                for cb in range(4):
                    sl = pl.ds(cb * 16, 16)
                    acc = rows_v[q * _KPAD, sl]
                    for r in range(1, _KPAD):
                        acc = jnp.maximum(acc, rows_v[q * _KPAD + r, sl])
                    out_v[q, sl] = acc
                for cb in range(4, 8):
                    out_v[q, pl.ds(cb * 16, 16)] = zero
            pltpu.sync_copy(out_v, out_hbm.at[pl.ds(gbase, _GRP)])

        stage(0, 0)

        def body(i, carry):
            g0 = 2 * i
            stage(g0 + 1, 1)
            compute(g0, 0)

            @pl.when(g0 + 2 < ngrp)
            def _():
                stage(g0 + 2, 0)
            compute(g0 + 1, 1)
            return carry

        lax.fori_loop(0, ngrp // 2, body, 0)

    return sc_kernel


def _tail_kernel(nT, P, ptsR_ref, m_ref, sgn_ref, wcdT_ref, s1_ref, o1_ref,
                 wencT_ref, s2_ref, o2_ref, wclsT_ref, bcls_ref,
                 out_ref, accmax_ref, accsum_ref):
    t = pl.program_id(1)
    Xr = ptsR_ref[0]
    m = m_ref[0][:, :64]

    tcen = jnp.dot(Xr, wcdT_ref[...], preferred_element_type=jnp.float32)
    y = (m * sgn_ref[...] + tcen) * s1_ref[...] + o1_ref[...]
    e = jnp.where(y >= 0, y, 0.2 * y)

    z = jnp.dot(e, wencT_ref[...], preferred_element_type=jnp.float32)
    z = z * s2_ref[...] + o2_ref[...]
    z = 0.5 * z * (1.0 + lax.erf(z * 0.7071067811865475))

    tmax = jnp.max(z, axis=0, keepdims=True)
    tsum = jnp.sum(z, axis=0, keepdims=True)

    @pl.when(t == 0)
    def _():
        accmax_ref[...] = tmax
        accsum_ref[...] = tsum

    @pl.when(t > 0)
    def _():
        accmax_ref[...] = jnp.maximum(accmax_ref[...], tmax)
        accsum_ref[...] = accsum_ref[...] + tsum

    @pl.when(t == nT - 1)
    def _():
        feat = jnp.concatenate(
            [accmax_ref[...], accsum_ref[...] * (1.0 / P)], axis=1)
        out_ref[0] = (jnp.dot(feat, wclsT_ref[...],
                              preferred_element_type=jnp.float32)
                      + bcls_ref[...])


def _run_tail(pts, m, sgn, wcdT, s1, o1, wencT, s2, o2, wclsT, bcls,
              row_tile):
    B, P, _ = pts.shape
    nT = P // row_tile
    kern = functools.partial(_tail_kernel, nT, P)
    return pl.pallas_call(
        kern,
        grid=(B, nT),
        in_specs=[
            pl.BlockSpec((1, row_tile, 4), lambda b, t: (b, t, 0)),
            pl.BlockSpec((1, row_tile, 128), lambda b, t: (b, t, 0)),
            pl.BlockSpec((1, 64), lambda b, t: (0, 0)),
            pl.BlockSpec((4, 64), lambda b, t: (0, 0)),
            pl.BlockSpec((1, 64), lambda b, t: (0, 0)),
            pl.BlockSpec((1, 64), lambda b, t: (0, 0)),
            pl.BlockSpec((64, 128), lambda b, t: (0, 0)),
            pl.BlockSpec((1, 128), lambda b, t: (0, 0)),
            pl.BlockSpec((1, 128), lambda b, t: (0, 0)),
            pl.BlockSpec((256, 40), lambda b, t: (0, 0)),
            pl.BlockSpec((1, 40), lambda b, t: (0, 0)),
        ],
        out_specs=pl.BlockSpec((1, 1, 40), lambda b, t: (b, 0, 0)),
        out_shape=jax.ShapeDtypeStruct((B, 1, 40), jnp.float32),
        scratch_shapes=[
            pltpu.VMEM((1, 128), jnp.float32),
            pltpu.VMEM((1, 128), jnp.float32),
        ],
    )(pts, m, sgn, wcdT, s1, o1, wencT, s2, o2, wclsT, bcls)


def kernel(inputs, W_edge, bn1_gamma, bn1_beta, bn1_mean, bn1_var,
           W_enc, bn2_gamma, bn2_beta, bn2_mean, bn2_var, W_cls, b_cls):
    B = inputs.shape[0]
    pts = inputs.reshape(B, -1, inputs.shape[-1])[..., :4]
    P = pts.shape[1]

    wdT = W_edge[:, :4].T
    wcdT = (W_edge[:, 4:] - W_edge[:, :4]).T
    s1 = (bn1_gamma / jnp.sqrt(bn1_var + _EPS)).reshape(1, -1)
    o1 = (bn1_beta - bn1_mean * s1[0]).reshape(1, -1)
    sgn = jnp.where(s1 >= 0, 1.0, -1.0)
    wencT = W_enc.T
    s2 = (bn2_gamma / jnp.sqrt(bn2_var + _EPS)).reshape(1, -1)
    o2 = (bn2_beta - bn2_mean * s2[0]).reshape(1, -1)
    wclsT = W_cls.T
    bcls = b_cls.reshape(1, -1)

    row_tile = 2048 if P % 2048 == 0 else P
    sc_gather = _make_sc_gather_max(B * P)

    idx, s = _run_topk(pts, wdT, sgn, row_tile)
    m = sc_gather(s.reshape(B * P, 128), idx.reshape(-1))
    out = _run_tail(pts, m.reshape(B, P, 128), sgn, wcdT, s1, o1,
                    wencT, s2, o2, wclsT, bcls, row_tile)
    return out.reshape(B, 40)


# speedup vs baseline: 1.0741x; 1.0741x over previous
import functools

import jax
import jax.numpy as jnp
from jax import lax
from jax.experimental import pallas as pl
from jax.experimental.pallas import tpu as pltpu
from jax.experimental.pallas import tpu_sc as plsc

_EPS = 1e-5
_K = 20
_KPAD = 32
_NEG = -3.0e38
_NC = 2
_NS = 16
_NW = _NC * _NS
_GRP = 4


def _topk_kernel(P, K, ptsR_ref, ptsT_ref, wdT_ref, sgn_ref,
                 idx_ref, s_ref):
    b = pl.program_id(0)
    Xr = ptsR_ref[0]
    XT = ptsT_ref[0]
    R = Xr.shape[0]

    S = (jnp.dot(Xr, wdT_ref[...], preferred_element_type=jnp.float32)
         * sgn_ref[...])
    s_ref[0] = jnp.concatenate([S, jnp.zeros_like(S)], axis=1)

    rn = jnp.sum(Xr * Xr, axis=1, keepdims=True)
    cn = jnp.sum(XT * XT, axis=0, keepdims=True)
    D = 2.0 * jnp.dot(Xr, XT, preferred_element_type=jnp.float32) - rn - cn
    iota = lax.broadcasted_iota(jnp.int32, (R, P), 1)
    lane32 = lax.broadcasted_iota(jnp.int32, (R, _KPAD), 1)

    v0 = jnp.max(D, axis=1, keepdims=True)

    def body(i, carry):
        v_prev, idxs = carry
        cand = jnp.where(D == v_prev, iota, P)
        idx = jnp.min(cand, axis=1, keepdims=True)
        m = jnp.where(D >= v_prev, _NEG, D)
        v = jnp.max(m, axis=1, keepdims=True)
        idxs = jnp.where(lane32 == i - 1, idx, idxs)
        return v, idxs

    _, idxs = lax.fori_loop(1, K + 1,
                            body, (v0, jnp.zeros((R, _KPAD), jnp.int32)))
    idxs = jnp.where(lane32 < K, idxs, idxs[:, 0:1]) + b * P
    idx_ref[0] = idxs


def _run_topk(pts, wdT, sgn, row_tile):
    B, P, _ = pts.shape
    nT = P // row_tile
    ptsT = jnp.swapaxes(pts, 1, 2)
    kern = functools.partial(_topk_kernel, P, _K)
    return pl.pallas_call(
        kern,
        grid=(B, nT),
        in_specs=[
            pl.BlockSpec((1, row_tile, 4), lambda b, t: (b, t, 0)),
            pl.BlockSpec((1, 4, P), lambda b, t: (b, 0, 0)),
            pl.BlockSpec((4, 64), lambda b, t: (0, 0)),
            pl.BlockSpec((1, 64), lambda b, t: (0, 0)),
        ],
        out_specs=[
            pl.BlockSpec((1, row_tile, _KPAD), lambda b, t: (b, t, 0)),
            pl.BlockSpec((1, row_tile, 128), lambda b, t: (b, t, 0)),
        ],
        out_shape=[
            jax.ShapeDtypeStruct((B, P, _KPAD), jnp.int32),
            jax.ShapeDtypeStruct((B, P, 128), jnp.float32),
        ],
    )(pts, ptsT, wdT, sgn)


def _make_sc_gather_max(P):
    per_w = P // _NW
    ngrp = per_w // _GRP
    mesh = plsc.VectorSubcoreMesh(core_axis_name="c", subcore_axis_name="s")

    @functools.partial(
        pl.kernel, mesh=mesh,
        out_type=jax.ShapeDtypeStruct((P, 128), jnp.float32),
        scratch_types=[
            pltpu.VMEM((_GRP * _KPAD,), jnp.int32),
            pltpu.VMEM((_GRP * _KPAD,), jnp.int32),
            pltpu.VMEM((_GRP * _KPAD, 128), jnp.float32),
            pltpu.VMEM((_GRP * _KPAD, 128), jnp.float32),
            pltpu.VMEM((_GRP, 128), jnp.float32),
            pltpu.SemaphoreType.DMA,
            pltpu.SemaphoreType.DMA,
        ],
    )
    def sc_kernel(s_hbm, idx_hbm, out_hbm, idx_v0, idx_v1,
                  rows_v0, rows_v1, out_v, sem0, sem1):
        wid = lax.axis_index("s") * _NC + lax.axis_index("c")
        base_pt = wid * per_w
        idx_vs = (idx_v0, idx_v1)
        rows_vs = (rows_v0, rows_v1)
        sems = (sem0, sem1)

        def stage(g, slot):
            gbase = base_pt + g * _GRP
            pltpu.sync_copy(idx_hbm.at[pl.ds(gbase * _KPAD, _GRP * _KPAD)],
                            idx_vs[slot])
            pltpu.async_copy(s_hbm.at[idx_vs[slot]], rows_vs[slot],
                             sems[slot])

        def compute(g, slot):
            gbase = base_pt + g * _GRP
            pltpu.make_async_copy(s_hbm.at[idx_vs[slot]], rows_vs[slot],
                                  sems[slot]).wait()
            rows_v = rows_vs[slot]
            zero = jnp.zeros((16,), jnp.float32)
            for q in range(_GRP):
                for cb in range(4):
                    sl = pl.ds(cb * 16, 16)
                    acc = rows_v[q * _KPAD, sl]
                    for r in range(1, _KPAD):
                        acc = jnp.maximum(acc, rows_v[q * _KPAD + r, sl])
                    out_v[q, sl] = acc
                for cb in range(4, 8):
                    out_v[q, pl.ds(cb * 16, 16)] = zero
            pltpu.sync_copy(out_v, out_hbm.at[pl.ds(gbase, _GRP)])

        stage(0, 0)

        def body(i, carry):
            g0 = 2 * i
            stage(g0 + 1, 1)
            compute(g0, 0)

            @pl.when(g0 + 2 < ngrp)
            def _():
                stage(g0 + 2, 0)
            compute(g0 + 1, 1)
            return carry

        lax.fori_loop(0, ngrp // 2, body, 0)

    return sc_kernel


def _tail_kernel(nT, P, ptsR_ref, m_ref, sgn_ref, wcdT_ref, s1_ref, o1_ref,
                 wencT_ref, s2_ref, o2_ref, wclsT_ref, bcls_ref,
                 out_ref, accmax_ref, accsum_ref):
    t = pl.program_id(1)
    Xr = ptsR_ref[0]
    m = m_ref[0][:, :64]

    tcen = jnp.dot(Xr, wcdT_ref[...], preferred_element_type=jnp.float32)
    y = (m * sgn_ref[...] + tcen) * s1_ref[...] + o1_ref[...]
    e = jnp.where(y >= 0, y, 0.2 * y)

    z = jnp.dot(e, wencT_ref[...], preferred_element_type=jnp.float32)
    z = z * s2_ref[...] + o2_ref[...]
    z = 0.5 * z * (1.0 + lax.erf(z * 0.7071067811865475))

    tmax = jnp.max(z, axis=0, keepdims=True)
    tsum = jnp.sum(z, axis=0, keepdims=True)

    @pl.when(t == 0)
    def _():
        accmax_ref[...] = tmax
        accsum_ref[...] = tsum

    @pl.when(t > 0)
    def _():
        accmax_ref[...] = jnp.maximum(accmax_ref[...], tmax)
        accsum_ref[...] = accsum_ref[...] + tsum

    @pl.when(t == nT - 1)
    def _():
        feat = jnp.concatenate(
            [accmax_ref[...], accsum_ref[...] * (1.0 / P)], axis=1)
        out_ref[0] = (jnp.dot(feat, wclsT_ref[...],
                              preferred_element_type=jnp.float32)
                      + bcls_ref[...])


def _run_tail(pts, m, sgn, wcdT, s1, o1, wencT, s2, o2, wclsT, bcls,
              row_tile):
    B, P, _ = pts.shape
    nT = P // row_tile
    kern = functools.partial(_tail_kernel, nT, P)
    return pl.pallas_call(
        kern,
        grid=(B, nT),
        in_specs=[
            pl.BlockSpec((1, row_tile, 4), lambda b, t: (b, t, 0)),
            pl.BlockSpec((1, row_tile, 128), lambda b, t: (b, t, 0)),
            pl.BlockSpec((1, 64), lambda b, t: (0, 0)),
            pl.BlockSpec((4, 64), lambda b, t: (0, 0)),
            pl.BlockSpec((1, 64), lambda b, t: (0, 0)),
            pl.BlockSpec((1, 64), lambda b, t: (0, 0)),
            pl.BlockSpec((64, 128), lambda b, t: (0, 0)),
            pl.BlockSpec((1, 128), lambda b, t: (0, 0)),
            pl.BlockSpec((1, 128), lambda b, t: (0, 0)),
            pl.BlockSpec((256, 40), lambda b, t: (0, 0)),
            pl.BlockSpec((1, 40), lambda b, t: (0, 0)),
        ],
        out_specs=pl.BlockSpec((1, 1, 40), lambda b, t: (b, 0, 0)),
        out_shape=jax.ShapeDtypeStruct((B, 1, 40), jnp.float32),
        scratch_shapes=[
            pltpu.VMEM((1, 128), jnp.float32),
            pltpu.VMEM((1, 128), jnp.float32),
        ],
    )(pts, m, sgn, wcdT, s1, o1, wencT, s2, o2, wclsT, bcls)


def kernel(inputs, W_edge, bn1_gamma, bn1_beta, bn1_mean, bn1_var,
           W_enc, bn2_gamma, bn2_beta, bn2_mean, bn2_var, W_cls, b_cls):
    B = inputs.shape[0]
    pts = inputs.reshape(B, -1, inputs.shape[-1])[..., :4]
    P = pts.shape[1]

    wdT = W_edge[:, :4].T
    wcdT = (W_edge[:, 4:] - W_edge[:, :4]).T
    s1 = (bn1_gamma / jnp.sqrt(bn1_var + _EPS)).reshape(1, -1)
    o1 = (bn1_beta - bn1_mean * s1[0]).reshape(1, -1)
    sgn = jnp.where(s1 >= 0, 1.0, -1.0)
    wencT = W_enc.T
    s2 = (bn2_gamma / jnp.sqrt(bn2_var + _EPS)).reshape(1, -1)
    o2 = (bn2_beta - bn2_mean * s2[0]).reshape(1, -1)
    wclsT = W_cls.T
    bcls = b_cls.reshape(1, -1)

    row_tile = 2048 if P % 2048 == 0 else P
    sc_gather = _make_sc_gather_max(P)

    logits = []
    for b in range(B):
        pts_b = pts[b:b + 1]
        idx_b, s_b = _run_topk(pts_b, wdT, sgn, row_tile)
        m_b = sc_gather(s_b.reshape(P, 128), idx_b.reshape(-1))
        logits.append(_run_tail(pts_b, m_b.reshape(1, P, 128), sgn, wcdT,
                                s1, o1, wencT, s2, o2, wclsT, bcls,
                                row_tile))
    return jnp.concatenate(logits, axis=0).reshape(B, 40)
